# Initial kernel scaffold; baseline (speedup 1.0000x reference)
import functools

import jax
import jax.numpy as jnp
from jax import lax
from jax.experimental import pallas as pl
from jax.experimental.pallas import tpu as pltpu
from jax.experimental.pallas import tpu_sc as plsc

NC, NS, LANES = 2, 16, 16
CH = 80


def _vmesh():
    return plsc.VectorSubcoreMesh(core_axis_name="c", subcore_axis_name="s")


def _fill(ref, rows, width, value):
    @pl.loop(0, rows)
    def _(i):
        @pl.loop(0, width // LANES)
        def _(j):
            ref[pl.ds(i, 1), pl.ds(j * LANES, LANES)] = jnp.full(
                (1, LANES), value, jnp.float32)


def _sc_degree(edge_index, n_nodes):
    e = edge_index.shape[1]
    per_tile = e // (NC * NS)
    chunks = per_tile // CH
    rows_pt = n_nodes // NS

    @functools.partial(
        pl.kernel,
        out_type=jax.ShapeDtypeStruct((NC, n_nodes, LANES), jnp.float32),
        mesh=_vmesh(),
        scratch_types=[
            pltpu.VMEM((CH,), jnp.int32),
            pltpu.VMEM((CH, LANES), jnp.float32),
            pltpu.VMEM((rows_pt, LANES), jnp.float32),
            pltpu.VMEM_SHARED((n_nodes, LANES), jnp.float32),
        ],
    )
    def k(ei_hbm, out_hbm, idx_v, ones_v, z_v, acc_sh):
        c = lax.axis_index("c")
        s = lax.axis_index("s")
        _fill(ones_v, CH, LANES, 1.0)
        _fill(z_v, rows_pt, LANES, 0.0)
        pltpu.sync_copy(z_v, acc_sh.at[pl.ds(s * rows_pt, rows_pt), :])
        plsc.subcore_barrier()
        base = (c * NS + s) * per_tile

        @pl.loop(0, chunks)
        def _(i):
            pltpu.sync_copy(ei_hbm.at[1, pl.ds(base + i * CH, CH)], idx_v)
            pltpu.sync_copy(ones_v, acc_sh.at[idx_v], add=True)

        plsc.subcore_barrier()
        pltpu.sync_copy(acc_sh.at[pl.ds(s * rows_pt, rows_pt), :],
                        out_hbm.at[c, pl.ds(s * rows_pt, rows_pt), :])

    return k(edge_index)


def _dis_from_deg(deg):
    return jnp.where(deg > 0, lax.rsqrt(jnp.where(deg > 0, deg, 1.0)), 0.0)


def _tc_prep(x, mask, degp):
    n, d = x.shape
    tw = d + LANES
    bn = 1000

    def body(x_ref, m_ref, degp_ref, t1_ref, t2_ref):
        deg = degp_ref[0, :, 0:1] + degp_ref[1, :, 0:1]
        dis = _dis_from_deg(deg)
        xm = x_ref[...] * m_ref[...]
        t1_ref[:, :d] = dis * xm
        t1_ref[:, d:] = jnp.broadcast_to(dis, (bn, LANES))
        t2_ref[:, :d] = dis * m_ref[...]
        t2_ref[:, d:] = jnp.broadcast_to(dis, (bn, LANES))

    return pl.pallas_call(
        body,
        grid=(n // bn,),
        in_specs=[
            pl.BlockSpec((bn, d), lambda i: (i, 0)),
            pl.BlockSpec((bn, d), lambda i: (i, 0)),
            pl.BlockSpec((NC, bn, LANES), lambda i: (0, i, 0)),
        ],
        out_specs=[
            pl.BlockSpec((bn, tw), lambda i: (i, 0)),
            pl.BlockSpec((bn, tw), lambda i: (i, 0)),
        ],
        out_shape=[jax.ShapeDtypeStruct((n, tw), jnp.float32)] * 2,
    )(x, mask, degp)


def _sc_scatter(t1, t2, edge_index, n_nodes):
    e = edge_index.shape[1]
    tw = t1.shape[1]
    per_tile = e // NS
    chunks = per_tile // CH
    rows_pt = n_nodes // NS
    zr = 125

    @functools.partial(
        pl.kernel,
        out_type=jax.ShapeDtypeStruct((NC, n_nodes, tw), jnp.float32),
        mesh=_vmesh(),
        scratch_types=[
            pltpu.VMEM((2, CH), jnp.int32),
            pltpu.VMEM((2, CH), jnp.int32),
            pltpu.VMEM((2, CH, tw), jnp.float32),
            pltpu.VMEM((zr, tw), jnp.float32),
            pltpu.VMEM_SHARED((n_nodes, tw), jnp.float32),
            pltpu.SemaphoreType.DMA,
            pltpu.SemaphoreType.DMA,
        ],
    )
    def k(t1_hbm, t2_hbm, ei_hbm, out_hbm, colv, rowv, rows_v, z_v, acc_sh,
          sem0, sem1):
        c = lax.axis_index("c")
        s = lax.axis_index("s")
        _fill(z_v, zr, tw, 0.0)

        @pl.loop(0, rows_pt // zr)
        def _(t):
            pltpu.sync_copy(z_v, acc_sh.at[pl.ds(s * rows_pt + t * zr, zr), :])

        plsc.subcore_barrier()
        base = s * per_tile

        def run(table_hbm):
            def stage_fire(chunk, slot, sem):
                off = base + chunk * CH
                pltpu.sync_copy(ei_hbm.at[1, pl.ds(off, CH)], colv.at[slot])
                pltpu.sync_copy(ei_hbm.at[0, pl.ds(off, CH)], rowv.at[slot])
                pltpu.async_copy(table_hbm.at[colv.at[slot]],
                                 rows_v.at[slot], sem)

            def drain(slot, sem):
                pltpu.make_async_copy(table_hbm.at[colv.at[slot]],
                                      rows_v.at[slot], sem).wait()
                pltpu.sync_copy(rows_v.at[slot], acc_sh.at[rowv.at[slot]],
                                add=True)

            stage_fire(0, 0, sem0)
            stage_fire(1, 1, sem1)

            @pl.loop(0, chunks // 2)
            def _(g):
                k0 = 2 * g
                drain(0, sem0)

                @pl.when(k0 + 2 < chunks)
                def _():
                    stage_fire(k0 + 2, 0, sem0)

                drain(1, sem1)

                @pl.when(k0 + 3 < chunks)
                def _():
                    stage_fire(k0 + 3, 1, sem1)

        @pl.when(c == 0)
        def _():
            run(t1_hbm)

        @pl.when(c == 1)
        def _():
            run(t2_hbm)

        plsc.subcore_barrier()
        pltpu.sync_copy(acc_sh.at[pl.ds(s * rows_pt, rows_pt), :],
                        out_hbm.at[c, pl.ds(s * rows_pt, rows_pt), :])

    return k(t1, t2, edge_index)


def _tc_final(o, degp, w, b):
    _, n, tw = o.shape
    d = w.shape[0]
    bn = 1000

    def body(o_ref, degp_ref, w_ref, b_ref, out_ref):
        deg = degp_ref[0, :, 0:1] + degp_ref[1, :, 0:1]
        dis = _dis_from_deg(deg)
        a1 = o_ref[0]
        a2 = o_ref[1]
        num = dis * a1[:, :d]
        den = dis * a2[:, :d]
        dhat = dis * a1[:, d:d + 1]
        partial = dhat * (num / (den + 1e-10))
        acc = lax.dot_general(partial, w_ref[...], (((1,), (1,)), ((), ())),
                              precision=lax.Precision.HIGHEST,
                              preferred_element_type=jnp.float32)
        out_ref[...] = acc + b_ref[...]

    return pl.pallas_call(
        body,
        grid=(n // bn,),
        in_specs=[
            pl.BlockSpec((NC, bn, tw), lambda i: (0, i, 0)),
            pl.BlockSpec((NC, bn, LANES), lambda i: (0, i, 0)),
            pl.BlockSpec((d, d), lambda i: (0, 0)),
            pl.BlockSpec((1, d), lambda i: (0, 0)),
        ],
        out_specs=pl.BlockSpec((bn, d), lambda i: (i, 0)),
        out_shape=jax.ShapeDtypeStruct((n, d), jnp.float32),
    )(o, degp, w, b)


def kernel(x, edge_index, mask, W, b):
    n, _ = x.shape
    degp = _sc_degree(edge_index, n)
    t1, t2 = _tc_prep(x, mask, degp)
    o = _sc_scatter(t1, t2, edge_index, n)
    return _tc_final(o, degp, W, b.reshape(1, -1))


# speedup vs baseline: 1.2689x; 1.2689x over previous
import jax
import jax.numpy as jnp
from jax import lax
from jax.experimental import pallas as pl
from jax.experimental.pallas import tpu as pltpu

EB = 512


def _deg_kernel(ei_ref, deg_ref):
    @pl.when(pl.program_id(0) == 0)
    def _():
        deg_ref[...] = jnp.zeros_like(deg_ref)

    def body(i, _):
        c = ei_ref[1, i]
        deg_ref[pl.ds(c, 1), :] += 1.0
        return 0

    lax.fori_loop(0, EB, body, 0)


def _agg_kernel(ei_ref, x_ref, m_ref, deg_ref, num_ref, den_ref, dh_ref,
                dis_ref):
    @pl.when(pl.program_id(0) == 0)
    def _():
        num_ref[...] = jnp.zeros_like(num_ref)
        den_ref[...] = jnp.zeros_like(den_ref)
        dh_ref[...] = jnp.zeros_like(dh_ref)
        deg = deg_ref[...]
        dis_ref[...] = jnp.where(
            deg > 0, lax.rsqrt(jnp.where(deg > 0, deg, 1.0)), 0.0)

    def body(i, _):
        r = ei_ref[0, i]
        c = ei_ref[1, i]
        w = dis_ref[pl.ds(c, 1), :]
        m_row = m_ref[pl.ds(c, 1), :]
        num_ref[pl.ds(r, 1), :] += w * (x_ref[pl.ds(c, 1), :] * m_row)
        den_ref[pl.ds(r, 1), :] += w * m_row
        dh_ref[pl.ds(r, 1), :] += w
        return 0

    lax.fori_loop(0, EB, body, 0)


def _final_kernel(num_ref, den_ref, dh_ref, deg_ref, w_ref, b_ref, out_ref):
    deg = deg_ref[:, 0:1]
    dis = jnp.where(deg > 0, lax.rsqrt(jnp.where(deg > 0, deg, 1.0)), 0.0)
    num = dis * num_ref[...]
    den = dis * den_ref[...]
    dhat = dis * dh_ref[:, 0:1]
    partial = dhat * (num / (den + 1e-10))
    acc = lax.dot_general(partial, w_ref[...], (((1,), (1,)), ((), ())),
                          precision=lax.Precision.HIGHEST,
                          preferred_element_type=jnp.float32)
    out_ref[...] = acc + b_ref[...]


def kernel(x, edge_index, mask, W, b):
    n, d = x.shape
    e = edge_index.shape[1]
    steps = e // EB

    ei_spec = pl.BlockSpec((2, EB), lambda i: (0, i),
                           memory_space=pltpu.SMEM)
    full = pl.BlockSpec((n, d), lambda i: (0, 0))

    deg = pl.pallas_call(
        _deg_kernel,
        grid=(steps,),
        in_specs=[ei_spec],
        out_specs=full,
        out_shape=jax.ShapeDtypeStruct((n, d), jnp.float32),
    )(edge_index)

    num, den, dh = pl.pallas_call(
        _agg_kernel,
        grid=(steps,),
        in_specs=[ei_spec, full, full, full],
        out_specs=[full, full, full],
        out_shape=[jax.ShapeDtypeStruct((n, d), jnp.float32)] * 3,
        scratch_shapes=[pltpu.VMEM((n, d), jnp.float32)],
    )(edge_index, x, mask, deg)

    bn = 1000
    return pl.pallas_call(
        _final_kernel,
        grid=(n // bn,),
        in_specs=[
            pl.BlockSpec((bn, d), lambda i: (i, 0)),
            pl.BlockSpec((bn, d), lambda i: (i, 0)),
            pl.BlockSpec((bn, d), lambda i: (i, 0)),
            pl.BlockSpec((bn, d), lambda i: (i, 0)),
            pl.BlockSpec((d, d), lambda i: (0, 0)),
            pl.BlockSpec((1, d), lambda i: (0, 0)),
        ],
        out_specs=pl.BlockSpec((bn, d), lambda i: (i, 0)),
        out_shape=jax.ShapeDtypeStruct((n, d), jnp.float32),
    )(num, den, dh, deg, W, b.reshape(1, -1))


# speedup vs baseline: 1.3770x; 1.0852x over previous
import jax
import jax.numpy as jnp
from jax import lax
from jax.experimental import pallas as pl
from jax.experimental.pallas import tpu as pltpu

EB = 512


def _deg_kernel(ei_ref, deg_ref):
    @pl.when(pl.program_id(0) == 0)
    def _():
        deg_ref[...] = jnp.zeros_like(deg_ref)

    def body(i, _):
        c = ei_ref[1, i]
        deg_ref[pl.ds(c, 1), :] += 1.0
        return 0

    lax.fori_loop(0, EB, body, 0)


def _wcat_kernel(x_ref, m_ref, deg_ref, wcat_ref):
    d = x_ref.shape[1]
    deg = deg_ref[...]
    dis = jnp.where(deg > 0, lax.rsqrt(jnp.where(deg > 0, deg, 1.0)), 0.0)
    wcat_ref[:, :d] = dis * x_ref[...] * m_ref[...]
    wcat_ref[:, d:2 * d] = dis * m_ref[...]
    wcat_ref[:, 2 * d:] = dis


def _agg_kernel(ei_ref, wcat_ref, acc_ref):
    @pl.when(pl.program_id(0) == 0)
    def _():
        acc_ref[...] = jnp.zeros_like(acc_ref)

    def body(i, _):
        r = ei_ref[0, i]
        c = ei_ref[1, i]
        acc_ref[pl.ds(r, 1), :] += wcat_ref[pl.ds(c, 1), :]
        return 0

    lax.fori_loop(0, EB, body, 0)


def _final_kernel(acc_ref, deg_ref, w_ref, b_ref, out_ref):
    d = w_ref.shape[0]
    deg = deg_ref[:, 0:1]
    dis = jnp.where(deg > 0, lax.rsqrt(jnp.where(deg > 0, deg, 1.0)), 0.0)
    num = dis * acc_ref[:, :d]
    den = dis * acc_ref[:, d:2 * d]
    dhat = dis * acc_ref[:, 2 * d:2 * d + 1]
    partial = dhat * (num / (den + 1e-10))
    acc = lax.dot_general(partial, w_ref[...], (((1,), (1,)), ((), ())),
                          precision=lax.Precision.HIGHEST,
                          preferred_element_type=jnp.float32)
    out_ref[...] = acc + b_ref[...]


def kernel(x, edge_index, mask, W, b):
    n, d = x.shape
    e = edge_index.shape[1]
    steps = e // EB

    ei_spec = pl.BlockSpec((2, EB), lambda i: (0, i),
                           memory_space=pltpu.SMEM)
    full = pl.BlockSpec((n, d), lambda i: (0, 0))
    full3 = pl.BlockSpec((n, 3 * d), lambda i: (0, 0))

    deg = pl.pallas_call(
        _deg_kernel,
        grid=(steps,),
        in_specs=[ei_spec],
        out_specs=full,
        out_shape=jax.ShapeDtypeStruct((n, d), jnp.float32),
    )(edge_index)

    bn = 1000
    wcat = pl.pallas_call(
        _wcat_kernel,
        grid=(n // bn,),
        in_specs=[
            pl.BlockSpec((bn, d), lambda i: (i, 0)),
            pl.BlockSpec((bn, d), lambda i: (i, 0)),
            pl.BlockSpec((bn, d), lambda i: (i, 0)),
        ],
        out_specs=pl.BlockSpec((bn, 3 * d), lambda i: (i, 0)),
        out_shape=jax.ShapeDtypeStruct((n, 3 * d), jnp.float32),
    )(x, mask, deg)

    acc = pl.pallas_call(
        _agg_kernel,
        grid=(steps,),
        in_specs=[ei_spec, full3],
        out_specs=full3,
        out_shape=jax.ShapeDtypeStruct((n, 3 * d), jnp.float32),
    )(edge_index, wcat)

    return pl.pallas_call(
        _final_kernel,
        grid=(n // bn,),
        in_specs=[
            pl.BlockSpec((bn, 3 * d), lambda i: (i, 0)),
            pl.BlockSpec((bn, d), lambda i: (i, 0)),
            pl.BlockSpec((d, d), lambda i: (0, 0)),
            pl.BlockSpec((1, d), lambda i: (0, 0)),
        ],
        out_specs=pl.BlockSpec((bn, d), lambda i: (i, 0)),
        out_shape=jax.ShapeDtypeStruct((n, d), jnp.float32),
    )(acc, deg, W, b.reshape(1, -1))


# speedup vs baseline: 2.0144x; 1.4629x over previous
import jax
import jax.numpy as jnp
from jax import lax
from jax.experimental import pallas as pl
from jax.experimental.pallas import tpu as pltpu

EB = 512


def _deg_kernel(ei_ref, deg_ref, deg1_ref):
    @pl.when(pl.program_id(0) == 0)
    def _():
        deg_ref[...] = jnp.zeros_like(deg_ref)
        deg1_ref[...] = jnp.zeros_like(deg1_ref)

    def body(i, _):
        c0 = ei_ref[1, 2 * i]
        c1 = ei_ref[1, 2 * i + 1]
        deg_ref[pl.ds(c0, 1), :] += 1.0
        deg1_ref[pl.ds(c1, 1), :] += 1.0
        return 0

    lax.fori_loop(0, EB // 2, body, 0)


def _wcat_kernel(x_ref, m_ref, deg_ref, deg1_ref, wcat_ref):
    d = x_ref.shape[1]
    deg = deg_ref[...] + deg1_ref[...]
    dis = jnp.where(deg > 0, lax.rsqrt(jnp.where(deg > 0, deg, 1.0)), 0.0)
    wcat_ref[:, :d] = dis * x_ref[...] * m_ref[...]
    wcat_ref[:, d:2 * d] = dis * m_ref[...]
    wcat_ref[:, 2 * d:] = dis


def _agg_kernel(ei_ref, wcat_ref, acc_ref, acc1_ref):
    @pl.when(pl.program_id(0) == 0)
    def _():
        acc_ref[...] = jnp.zeros_like(acc_ref)
        acc1_ref[...] = jnp.zeros_like(acc1_ref)

    def body(i, _):
        r0 = ei_ref[0, 2 * i]
        c0 = ei_ref[1, 2 * i]
        r1 = ei_ref[0, 2 * i + 1]
        c1 = ei_ref[1, 2 * i + 1]
        acc_ref[pl.ds(r0, 1), :] += wcat_ref[pl.ds(c0, 1), :]
        acc1_ref[pl.ds(r1, 1), :] += wcat_ref[pl.ds(c1, 1), :]
        return 0

    lax.fori_loop(0, EB // 2, body, 0)


def _final_kernel(acc_ref, acc1_ref, deg_ref, deg1_ref, w_ref, b_ref,
                  out_ref):
    d = w_ref.shape[0]
    deg = deg_ref[:, 0:1] + deg1_ref[:, 0:1]
    dis = jnp.where(deg > 0, lax.rsqrt(jnp.where(deg > 0, deg, 1.0)), 0.0)
    a = acc_ref[...] + acc1_ref[...]
    num = dis * a[:, :d]
    den = dis * a[:, d:2 * d]
    dhat = dis * a[:, 2 * d:2 * d + 1]
    partial = dhat * (num / (den + 1e-10))
    acc = lax.dot_general(partial, w_ref[...], (((1,), (1,)), ((), ())),
                          precision=lax.Precision.HIGHEST,
                          preferred_element_type=jnp.float32)
    out_ref[...] = acc + b_ref[...]


def kernel(x, edge_index, mask, W, b):
    n, d = x.shape
    e = edge_index.shape[1]
    steps = e // EB

    ei_spec = pl.BlockSpec((2, EB), lambda i: (0, i),
                           memory_space=pltpu.SMEM)
    full = pl.BlockSpec((n, d), lambda i: (0, 0))
    full3 = pl.BlockSpec((n, 3 * d), lambda i: (0, 0))

    deg0, deg1 = pl.pallas_call(
        _deg_kernel,
        grid=(steps,),
        in_specs=[ei_spec],
        out_specs=[full, full],
        out_shape=[jax.ShapeDtypeStruct((n, d), jnp.float32)] * 2,
    )(edge_index)

    bn = 1000
    wcat = pl.pallas_call(
        _wcat_kernel,
        grid=(n // bn,),
        in_specs=[
            pl.BlockSpec((bn, d), lambda i: (i, 0)),
            pl.BlockSpec((bn, d), lambda i: (i, 0)),
            pl.BlockSpec((bn, d), lambda i: (i, 0)),
            pl.BlockSpec((bn, d), lambda i: (i, 0)),
        ],
        out_specs=pl.BlockSpec((bn, 3 * d), lambda i: (i, 0)),
        out_shape=jax.ShapeDtypeStruct((n, 3 * d), jnp.float32),
    )(x, mask, deg0, deg1)

    acc0, acc1 = pl.pallas_call(
        _agg_kernel,
        grid=(steps,),
        in_specs=[ei_spec, full3],
        out_specs=[full3, full3],
        out_shape=[jax.ShapeDtypeStruct((n, 3 * d), jnp.float32)] * 2,
    )(edge_index, wcat)

    return pl.pallas_call(
        _final_kernel,
        grid=(n // bn,),
        in_specs=[
            pl.BlockSpec((bn, 3 * d), lambda i: (i, 0)),
            pl.BlockSpec((bn, 3 * d), lambda i: (i, 0)),
            pl.BlockSpec((bn, d), lambda i: (i, 0)),
            pl.BlockSpec((bn, d), lambda i: (i, 0)),
            pl.BlockSpec((d, d), lambda i: (0, 0)),
            pl.BlockSpec((1, d), lambda i: (0, 0)),
        ],
        out_specs=pl.BlockSpec((bn, d), lambda i: (i, 0)),
        out_shape=jax.ShapeDtypeStruct((n, d), jnp.float32),
    )(acc0, acc1, deg0, deg1, W, b.reshape(1, -1))


# speedup vs baseline: 2.0313x; 1.0084x over previous
import jax
import jax.numpy as jnp
from jax import lax
from jax.experimental import pallas as pl
from jax.experimental.pallas import tpu as pltpu

EB = 2560


def _deg_kernel(ei_ref, deg_ref, deg1_ref):
    @pl.when(pl.program_id(0) == 0)
    def _():
        deg_ref[...] = jnp.zeros_like(deg_ref)
        deg1_ref[...] = jnp.zeros_like(deg1_ref)

    def body(i, _):
        c0 = ei_ref[1, 2 * i]
        c1 = ei_ref[1, 2 * i + 1]
        deg_ref[pl.ds(c0, 1), :] += 1.0
        deg1_ref[pl.ds(c1, 1), :] += 1.0
        return 0

    lax.fori_loop(0, EB // 2, body, 0)


def _wcat_kernel(x_ref, m_ref, deg_ref, deg1_ref, wcat_ref):
    d = x_ref.shape[1]
    deg = deg_ref[...] + deg1_ref[...]
    dis = jnp.where(deg > 0, lax.rsqrt(jnp.where(deg > 0, deg, 1.0)), 0.0)
    wcat_ref[:, :d] = dis * x_ref[...] * m_ref[...]
    wcat_ref[:, d:2 * d] = dis * m_ref[...]
    wcat_ref[:, 2 * d:] = dis


def _agg_kernel(ei_ref, wcat_ref, acc_ref, acc1_ref):
    @pl.when(pl.program_id(0) == 0)
    def _():
        acc_ref[...] = jnp.zeros_like(acc_ref)
        acc1_ref[...] = jnp.zeros_like(acc1_ref)

    def body(i, _):
        r0 = ei_ref[0, 2 * i]
        c0 = ei_ref[1, 2 * i]
        r1 = ei_ref[0, 2 * i + 1]
        c1 = ei_ref[1, 2 * i + 1]
        acc_ref[pl.ds(r0, 1), :] += wcat_ref[pl.ds(c0, 1), :]
        acc1_ref[pl.ds(r1, 1), :] += wcat_ref[pl.ds(c1, 1), :]
        return 0

    lax.fori_loop(0, EB // 2, body, 0)


def _final_kernel(acc_ref, acc1_ref, deg_ref, deg1_ref, w_ref, b_ref,
                  out_ref):
    d = w_ref.shape[0]
    deg = deg_ref[:, 0:1] + deg1_ref[:, 0:1]
    dis = jnp.where(deg > 0, lax.rsqrt(jnp.where(deg > 0, deg, 1.0)), 0.0)
    a = acc_ref[...] + acc1_ref[...]
    num = dis * a[:, :d]
    den = dis * a[:, d:2 * d]
    dhat = dis * a[:, 2 * d:2 * d + 1]
    partial = dhat * (num / (den + 1e-10))
    acc = lax.dot_general(partial, w_ref[...], (((1,), (1,)), ((), ())),
                          precision=lax.Precision.HIGHEST,
                          preferred_element_type=jnp.float32)
    out_ref[...] = acc + b_ref[...]


def kernel(x, edge_index, mask, W, b):
    n, d = x.shape
    e = edge_index.shape[1]
    steps = e // EB

    ei_spec = pl.BlockSpec((2, EB), lambda i: (0, i),
                           memory_space=pltpu.SMEM)
    full = pl.BlockSpec((n, d), lambda i: (0, 0))
    full3 = pl.BlockSpec((n, 3 * d), lambda i: (0, 0))

    deg0, deg1 = pl.pallas_call(
        _deg_kernel,
        grid=(steps,),
        in_specs=[ei_spec],
        out_specs=[full, full],
        out_shape=[jax.ShapeDtypeStruct((n, d), jnp.float32)] * 2,
    )(edge_index)

    bn = 1000
    wcat = pl.pallas_call(
        _wcat_kernel,
        grid=(n // bn,),
        in_specs=[
            pl.BlockSpec((bn, d), lambda i: (i, 0)),
            pl.BlockSpec((bn, d), lambda i: (i, 0)),
            pl.BlockSpec((bn, d), lambda i: (i, 0)),
            pl.BlockSpec((bn, d), lambda i: (i, 0)),
        ],
        out_specs=pl.BlockSpec((bn, 3 * d), lambda i: (i, 0)),
        out_shape=jax.ShapeDtypeStruct((n, 3 * d), jnp.float32),
    )(x, mask, deg0, deg1)

    acc0, acc1 = pl.pallas_call(
        _agg_kernel,
        grid=(steps,),
        in_specs=[ei_spec, full3],
        out_specs=[full3, full3],
        out_shape=[jax.ShapeDtypeStruct((n, 3 * d), jnp.float32)] * 2,
    )(edge_index, wcat)

    return pl.pallas_call(
        _final_kernel,
        grid=(n // bn,),
        in_specs=[
            pl.BlockSpec((bn, 3 * d), lambda i: (i, 0)),
            pl.BlockSpec((bn, 3 * d), lambda i: (i, 0)),
            pl.BlockSpec((bn, d), lambda i: (i, 0)),
            pl.BlockSpec((bn, d), lambda i: (i, 0)),
            pl.BlockSpec((d, d), lambda i: (0, 0)),
            pl.BlockSpec((1, d), lambda i: (0, 0)),
        ],
        out_specs=pl.BlockSpec((bn, d), lambda i: (i, 0)),
        out_shape=jax.ShapeDtypeStruct((n, d), jnp.float32),
    )(acc0, acc1, deg0, deg1, W, b.reshape(1, -1))
